# Initial kernel scaffold; baseline (speedup 1.0000x reference)
import jax
import jax.numpy as jnp
from jax.experimental import pallas as pl


def kernel(x, stc_enc, edge_index, batch, y, W_pre, b_pre, W_s, b_s, gin_W1, gin_b1, gin_W2, gin_b2, gcn_W, gcn_b, W_hp, b_hp, W_post, b_post, W_ro, b_ro):
    raise NotImplementedError("write your pallas kernel here")


# speedup vs baseline: 5.9979x; 5.9979x over previous
import functools

import jax
import jax.numpy as jnp
from jax import lax
from jax.experimental import pallas as pl
from jax.experimental.pallas import tpu as pltpu
from jax.experimental.pallas import tpu_sc as plsc

N = 10000
E = 320000
DF = 128
H = 128
NSE = 16
L = 3
G = 128
NC = 10

NSUB = 16
CH = 128
NCHUNK = 158
EP = NSUB * NCHUNK * CH
NP = 10112
STRIPE = NP // NSUB
DUMMY = N


@functools.cache
def _mesh():
    return plsc.VectorSubcoreMesh(
        core_axis_name="c", subcore_axis_name="s", num_cores=2, num_subcores=16
    )


def _sc_deg_body(dst_hbm, ones_hbm, z_hbm, out, ones_v, idxd, acc):
    c = lax.axis_index("c")
    s = lax.axis_index("s")
    pltpu.sync_copy(z_hbm, acc.at[pl.ds(s * STRIPE, STRIPE)])
    pltpu.sync_copy(ones_hbm, ones_v)
    plsc.subcore_barrier()

    @pl.loop(0, NCHUNK // 2)
    def _(j):
        k = 2 * j + c
        pltpu.sync_copy(dst_hbm.at[s, k], idxd.at[0])
        pltpu.sync_copy(ones_v, acc.at[idxd.at[0]], add=True)

    plsc.subcore_barrier()
    sl = pl.ds(s * STRIPE, STRIPE)
    pltpu.sync_copy(acc.at[sl], out.at[c, sl])


@functools.cache
def _make_sc_deg():
    return functools.partial(
        pl.kernel,
        out_type=jax.ShapeDtypeStruct((2, NP, H), jnp.float32),
        mesh=_mesh(),
        scratch_types=[
            pltpu.VMEM((CH, H), jnp.float32),
            pltpu.VMEM((1, CH), jnp.int32),
            pltpu.VMEM_SHARED((NP, H), jnp.float32),
        ],
    )(_sc_deg_body)


def _sc_agg_body(table_hbm, src_hbm, dst_hbm, z_hbm, out, idxs, idxd, rows,
                 acc):
    c = lax.axis_index("c")
    s = lax.axis_index("s")
    pltpu.sync_copy(z_hbm, acc.at[pl.ds(s * STRIPE, STRIPE)])
    plsc.subcore_barrier()

    @pl.loop(0, NCHUNK // 2)
    def _(j):
        k = 2 * j + c
        pltpu.sync_copy(src_hbm.at[s, k], idxs.at[0])
        pltpu.sync_copy(dst_hbm.at[s, k], idxd.at[0])
        pltpu.sync_copy(table_hbm.at[idxs.at[0]], rows)
        pltpu.sync_copy(rows, acc.at[idxd.at[0]], add=True)

    plsc.subcore_barrier()
    sl = pl.ds(s * STRIPE, STRIPE)
    pltpu.sync_copy(acc.at[sl], out.at[c, sl])


@functools.cache
def _make_sc_agg():
    return functools.partial(
        pl.kernel,
        out_type=jax.ShapeDtypeStruct((2, NP, H), jnp.float32),
        mesh=_mesh(),
        scratch_types=[
            pltpu.VMEM((1, CH), jnp.int32),
            pltpu.VMEM((1, CH), jnp.int32),
            pltpu.VMEM((CH, H), jnp.float32),
            pltpu.VMEM_SHARED((NP, H), jnp.float32),
        ],
    )(_sc_agg_body)


RB = 1264
_NG = NP // RB

def _dot(a, b):
    return jnp.dot(a, b, preferred_element_type=jnp.float32)


def _row_spec(w):
    return pl.BlockSpec((RB, w), lambda i: (i, 0))


def _prow_spec(w):
    return pl.BlockSpec((2, RB, w), lambda i: (0, i, 0))


def _full_spec(shape):
    return pl.BlockSpec(shape, lambda i: tuple(0 for _ in shape))


def _tc_init_body(x_ref, stc_ref, wpre, bpre, ws, bs, w1a, w1b, gw,
                  h_ref, s_ref, u_ref, xw_ref):
    h = _dot(x_ref[...], wpre[...]) + bpre[...]
    s = _dot(stc_ref[...], ws[...]) + bs[...]
    h_ref[...] = h
    s_ref[...] = s
    u_ref[...] = _dot(h, w1a[...]) + _dot(s, w1b[...])
    xw_ref[...] = _dot(s, gw[...])


def _tc_init(x, stc, wpre, bpre, ws, bs, w1a, w1b, gw):
    out = tuple(jax.ShapeDtypeStruct((NP, H), jnp.float32) for _ in range(4))
    return pl.pallas_call(
        _tc_init_body,
        grid=(_NG,),
        in_specs=[
            _row_spec(DF), _row_spec(NSE),
            _full_spec((DF, H)), _full_spec((1, H)),
            _full_spec((NSE, H)), _full_spec((1, H)),
            _full_spec((H, H)), _full_spec((H, H)), _full_spec((H, H)),
        ],
        out_specs=tuple(_row_spec(H) for _ in range(4)),
        out_shape=out,
    )(x, stc, wpre, bpre, ws, bs, w1a, w1b, gw)


def _tc_dinv_body(degp_ref, xw_ref, dinv_ref, xs_ref):
    deg = degp_ref[0][:, 0:1] + degp_ref[1][:, 0:1] + 1.0
    dinv = lax.rsqrt(deg)
    dinv_ref[...] = jnp.broadcast_to(dinv, dinv_ref.shape)
    xs_ref[...] = dinv * xw_ref[...]


def _tc_dinv(degp, xw):
    return pl.pallas_call(
        _tc_dinv_body,
        grid=(_NG,),
        in_specs=[_prow_spec(H), _row_spec(H)],
        out_specs=(_row_spec(16), _row_spec(H)),
        out_shape=(
            jax.ShapeDtypeStruct((NP, 16), jnp.float32),
            jax.ShapeDtypeStruct((NP, H), jnp.float32),
        ),
    )(degp, xw)


def _tc_layer_body(u_ref, aggu_ref, xw_ref, aggs_ref, dinv_ref,
                   b1, w2, b2, gb, wa, wb, bu, gwn,
                   h_ref, s_ref, un_ref, xwn_ref, xsn_ref):
    aggu = aggu_ref[0] + aggu_ref[1]
    aggs = aggs_ref[0] + aggs_ref[1]
    z = jax.nn.relu(u_ref[...] + aggu + b1[...])
    hn = jax.nn.relu(_dot(z, w2[...]) + b2[...])
    d = dinv_ref[...][:, 0:1]
    sn = jnp.tanh(d * aggs + (d * d) * xw_ref[...] + gb[...])
    h_ref[...] = hn
    s_ref[...] = sn
    un_ref[...] = _dot(hn, wa[...]) + _dot(sn, wb[...]) + bu[...]
    xwn = _dot(sn, gwn[...])
    xwn_ref[...] = xwn
    xsn_ref[...] = d * xwn


def _tc_layer(u, aggu, xw, aggs, dinv, b1, w2, b2, gb, wa, wb, bu, gwn):
    out = tuple(jax.ShapeDtypeStruct((NP, H), jnp.float32) for _ in range(5))
    return pl.pallas_call(
        _tc_layer_body,
        grid=(_NG,),
        in_specs=[
            _row_spec(H), _prow_spec(H), _row_spec(H), _prow_spec(H),
            _row_spec(16),
            _full_spec((1, H)), _full_spec((H, H)), _full_spec((1, H)),
            _full_spec((1, H)),
            _full_spec((H, H)), _full_spec((H, H)), _full_spec((1, H)),
            _full_spec((H, H)),
        ],
        out_specs=tuple(_row_spec(H) for _ in range(5)),
        out_shape=out,
    )(u, aggu, xw, aggs, dinv, b1, w2, b2, gb, wa, wb, bu, gwn)


def _tc_pool_body(hp_ref, batch_ref, wpost, bpost, wro, bro, out_ref, acc):
    i = pl.program_id(0)

    @pl.when(i == 0)
    def _():
        acc[...] = jnp.zeros_like(acc)

    b = batch_ref[0]
    brow = jnp.broadcast_to(b, (G, CH))
    gidx = lax.broadcasted_iota(jnp.int32, (G, CH), 0)
    mt = jnp.where(brow == gidx, 1.0, 0.0)
    acc[...] += _dot(mt, hp_ref[...])

    @pl.when(i == NP // CH - 1)
    def _():
        p = jax.nn.relu(_dot(acc[...], wpost[...]) + bpost[...])
        logits = _dot(p, wro[...]) + bro[...]
        m = jnp.max(logits, axis=1, keepdims=True)
        e = jnp.exp(logits - m)
        lse = jnp.log(jnp.sum(e, axis=1, keepdims=True)) + m
        out_ref[...] = logits - lse


def _tc_pool(hp, batch_r, wpost, bpost, wro, bro):
    return pl.pallas_call(
        _tc_pool_body,
        grid=(NP // CH,),
        in_specs=[
            pl.BlockSpec((CH, H), lambda i: (i, 0)),
            pl.BlockSpec((1, 1, CH), lambda i: (i, 0, 0)),
            _full_spec((H, H)), _full_spec((1, H)),
            _full_spec((H, NC)), _full_spec((1, NC)),
        ],
        out_specs=pl.BlockSpec((G, NC), lambda i: (0, 0)),
        out_shape=jax.ShapeDtypeStruct((G, NC), jnp.float32),
        scratch_shapes=[pltpu.VMEM((G, H), jnp.float32)],
    )(hp, batch_r, wpost, bpost, wro, bro)


@jax.jit
def _forward(x, stc_enc, edge_index, batch,
             W_pre, b_pre, W_s, b_s,
             gin_W1, gin_b1, gin_W2, gin_b2, gcn_W, gcn_b,
             W_hp, b_hp, W_post, b_post, W_ro, b_ro):
    f32 = jnp.float32
    src = jnp.pad(edge_index[0], (0, EP - E)).reshape(NSUB, NCHUNK, CH)
    dst = jnp.pad(edge_index[1], (0, EP - E),
                  constant_values=DUMMY).reshape(NSUB, NCHUNK, CH)
    xp = jnp.pad(x, ((0, NP - N), (0, 0)))
    stcp = jnp.pad(stc_enc, ((0, NP - N), (0, 0)))
    batch_r = jnp.pad(batch, (0, NP - N),
                      constant_values=G).reshape(NP // CH, 1, CH)

    z128 = jnp.zeros((STRIPE, H), f32)
    ones128 = jnp.ones((CH, H), f32)

    row = lambda v: v.reshape(1, -1)

    degp = _make_sc_deg()(dst, ones128, z128)

    h, s, u, xw = _tc_init(
        xp, stcp, W_pre, row(b_pre), W_s, row(b_s),
        gin_W1[0][:H], gin_W1[0][H:], gcn_W[0])
    dinv, xs = _tc_dinv(degp, xw)

    zero_bias = jnp.zeros((1, H), f32)
    for i in range(L):
        aggu = _make_sc_agg()(u, src, dst, z128)
        aggs = _make_sc_agg()(xs, src, dst, z128)
        last = i == L - 1
        if last:
            wa, wb, bu = W_hp[:H], W_hp[H:], row(b_hp)
            gwn = gcn_W[i]
        else:
            wa, wb, bu = gin_W1[i + 1][:H], gin_W1[i + 1][H:], zero_bias
            gwn = gcn_W[i + 1]
        h, s, u, xw, xs = _tc_layer(
            u, aggu, xw, aggs, dinv,
            row(gin_b1[i]), gin_W2[i], row(gin_b2[i]), row(gcn_b[i]),
            wa, wb, bu, gwn)

    return _tc_pool(u, batch_r, W_post, row(b_post), W_ro, row(b_ro))


def kernel(x, stc_enc, edge_index, batch, y, W_pre, b_pre, W_s, b_s,
           gin_W1, gin_b1, gin_W2, gin_b2, gcn_W, gcn_b,
           W_hp, b_hp, W_post, b_post, W_ro, b_ro):
    return _forward(x, stc_enc, edge_index, batch,
                    W_pre, b_pre, W_s, b_s,
                    gin_W1, gin_b1, gin_W2, gin_b2, gcn_W, gcn_b,
                    W_hp, b_hp, W_post, b_post, W_ro, b_ro)


# speedup vs baseline: 6.9023x; 1.1508x over previous
import dataclasses
import functools

import jax
import jax.numpy as jnp
from jax import lax
from jax.experimental import pallas as pl
from jax.experimental.pallas import tpu as pltpu
from jax.experimental.pallas import tpu_sc as plsc

N = 10000
E = 320000
DF = 128
H = 128
NSE = 16
L = 3
G = 128
NC = 10

NSUB = 16
CH = 128
NWA = 160
NWD = NWA // 2
NB = 2
EP = NSUB * NWA * CH
NP = 10240
STRIPE = NP // NSUB
DUMMY = N


@functools.cache
def _mesh():
    return plsc.VectorSubcoreMesh(
        core_axis_name="c", subcore_axis_name="s", num_cores=2, num_subcores=16
    )


def _sc_deghist_body(dst_hbm, z_hbm, out, dstv, hist):
    c = lax.axis_index("c")
    s = lax.axis_index("s")
    pltpu.sync_copy(z_hbm, hist)
    pltpu.sync_copy(dst_hbm.at[s, pl.ds(c * NWD, NWD)], dstv)
    ones = jnp.ones((16,), jnp.float32)

    @pl.loop(0, NWD)
    def _(r):
        @pl.loop(0, CH, step=16)
        def _(j):
            d = dstv[r, pl.ds(j, 16)]
            plsc.addupdate_scatter(hist, [d >> 7, d & 127], ones)

    pltpu.sync_copy(hist, out.at[c, s])


@functools.cache
def _make_sc_deghist():
    return functools.partial(
        pl.kernel,
        out_type=jax.ShapeDtypeStruct((2, NSUB, NP // CH, CH), jnp.float32),
        mesh=_mesh(),
        scratch_types=[
            pltpu.VMEM((NWD, CH), jnp.int32),
            pltpu.VMEM((NP // CH, CH), jnp.float32),
        ],
        compiler_params=dataclasses.replace(pltpu.CompilerParams(),
                                            needs_layout_passes=False),
    )(_sc_deghist_body)


BLK = 40

def _sc_agg_body(tab_hbm, src_hbm, dst_hbm, z_hbm, out,
                 srcv, dstv, r0, r1, acc, g0, g1, s0, s1):
    rows = (r0, r1)
    gsems = (g0, g1)
    ssems = (s0, s1)
    c = lax.axis_index("c")
    s = lax.axis_index("s")
    pltpu.sync_copy(z_hbm, acc.at[pl.ds(s * STRIPE, STRIPE)])
    plsc.subcore_barrier()

    for blk in range(NWA // BLK):
        pltpu.sync_copy(src_hbm.at[c, s, pl.ds(blk * BLK, BLK)], srcv)
        pltpu.sync_copy(dst_hbm.at[s, pl.ds(blk * BLK, BLK)], dstv)
        for b in range(NB):
            pltpu.async_copy(tab_hbm.at[srcv.at[b]], rows[b], gsems[b])

        @pl.loop(0, BLK - NB, step=NB)
        def _(g):
            for b in range(NB):
                k = g + b
                pltpu.make_async_copy(tab_hbm.at[srcv.at[b]], rows[b],
                                      gsems[b]).wait()
                pltpu.async_copy(rows[b], acc.at[dstv.at[k]], ssems[b],
                                 add=True)
                pltpu.make_async_copy(rows[b], acc.at[dstv.at[k]],
                                      ssems[b]).wait()
                pltpu.async_copy(tab_hbm.at[srcv.at[k + NB]], rows[b],
                                 gsems[b])

        for b in range(NB):
            k = BLK - NB + b
            pltpu.make_async_copy(tab_hbm.at[srcv.at[b]], rows[b],
                                  gsems[b]).wait()
            pltpu.sync_copy(rows[b], acc.at[dstv.at[k]], add=True)

    plsc.subcore_barrier()
    sl = pl.ds(s * STRIPE, STRIPE)
    pltpu.sync_copy(acc.at[sl], out.at[c, sl])


@functools.cache
def _make_sc_agg():
    return functools.partial(
        pl.kernel,
        out_type=jax.ShapeDtypeStruct((2, NP, H), jnp.float32),
        mesh=_mesh(),
        scratch_types=[
            pltpu.VMEM((BLK, CH), jnp.int32),
            pltpu.VMEM((BLK, CH), jnp.int32),
            pltpu.VMEM((CH, H), jnp.float32),
            pltpu.VMEM((CH, H), jnp.float32),
            pltpu.VMEM_SHARED((NP, H), jnp.float32),
            pltpu.SemaphoreType.DMA,
            pltpu.SemaphoreType.DMA,
            pltpu.SemaphoreType.DMA,
            pltpu.SemaphoreType.DMA,
        ],
    )(_sc_agg_body)


RB = 1280
_NG = NP // RB

def _dot(a, b):
    return jnp.dot(a, b, preferred_element_type=jnp.float32)


def _row_spec(w):
    return pl.BlockSpec((RB, w), lambda i: (i, 0))


def _prow_spec(w):
    return pl.BlockSpec((2, RB, w), lambda i: (0, i, 0))


def _full_spec(shape):
    return pl.BlockSpec(shape, lambda i: tuple(0 for _ in shape))


def _tc_init_body(x_ref, stc_ref, wpre, bpre, ws, bs, w1a, w1b, gw,
                  u_ref, xw_ref):
    h = _dot(x_ref[...], wpre[...]) + bpre[...]
    s = _dot(stc_ref[...], ws[...]) + bs[...]
    u_ref[...] = _dot(h, w1a[...]) + _dot(s, w1b[...])
    xw_ref[...] = _dot(s, gw[...])


def _tc_init(x, stc, wpre, bpre, ws, bs, w1a, w1b, gw):
    out = tuple(jax.ShapeDtypeStruct((NP, H), jnp.float32) for _ in range(2))
    return pl.pallas_call(
        _tc_init_body,
        grid=(_NG,),
        in_specs=[
            _row_spec(DF), _row_spec(NSE),
            _full_spec((DF, H)), _full_spec((1, H)),
            _full_spec((NSE, H)), _full_spec((1, H)),
            _full_spec((H, H)), _full_spec((H, H)), _full_spec((H, H)),
        ],
        out_specs=tuple(_row_spec(H) for _ in range(2)),
        out_shape=out,
    )(x, stc, wpre, bpre, ws, bs, w1a, w1b, gw)


def _tc_dinv_body(degw_ref, u_ref, xw_ref, dinv_ref, tab_ref):
    deg2d = jnp.sum(degw_ref[...], axis=(0, 1))
    dinv2d = lax.rsqrt(deg2d + 1.0)
    nsub = lax.broadcasted_iota(jnp.int32, (NP, NP // CH), 0) >> 7
    rsel = jnp.where(nsub == lax.broadcasted_iota(
        jnp.int32, (NP, NP // CH), 1), 1.0, 0.0)
    crow = _dot(rsel, dinv2d)
    nlan = lax.broadcasted_iota(jnp.int32, (NP, CH), 0) & (CH - 1)
    lsel = jnp.where(nlan == lax.broadcasted_iota(
        jnp.int32, (NP, CH), 1), 1.0, 0.0)
    dinv = jnp.sum(crow * lsel, axis=1, keepdims=True)
    dinv_ref[...] = jnp.broadcast_to(dinv, dinv_ref.shape)
    tab_ref[0] = u_ref[...]
    tab_ref[1] = dinv * xw_ref[...]


def _tc_dinv(degw, u, xw):
    return pl.pallas_call(
        _tc_dinv_body,
        grid=(1,),
        in_specs=[
            _full_spec((2, NSUB, NP // CH, CH)),
            _full_spec((NP, H)), _full_spec((NP, H)),
        ],
        out_specs=(_full_spec((NP, 16)), _full_spec((2, NP, H))),
        out_shape=(
            jax.ShapeDtypeStruct((NP, 16), jnp.float32),
            jax.ShapeDtypeStruct((2, NP, H), jnp.float32),
        ),
    )(degw, u, xw)


def _tc_layer_body(tab_ref, agg_ref, dinv_ref,
                   b1, w2, b2, gb, wa, wb, bu, gwn, tabn_ref):
    z = jax.nn.relu(tab_ref[0] + agg_ref[0] + b1[...])
    hn = jax.nn.relu(_dot(z, w2[...]) + b2[...])
    d = dinv_ref[...][:, 0:1]
    sn = jnp.tanh(d * (agg_ref[1] + tab_ref[1]) + gb[...])
    tabn_ref[0] = _dot(hn, wa[...]) + _dot(sn, wb[...]) + bu[...]
    tabn_ref[1] = d * _dot(sn, gwn[...])


def _tc_layer(tab, agg, dinv, b1, w2, b2, gb, wa, wb, bu, gwn):
    return pl.pallas_call(
        _tc_layer_body,
        grid=(_NG,),
        in_specs=[
            _prow_spec(H), _prow_spec(H), _row_spec(16),
            _full_spec((1, H)), _full_spec((H, H)), _full_spec((1, H)),
            _full_spec((1, H)),
            _full_spec((H, H)), _full_spec((H, H)), _full_spec((1, H)),
            _full_spec((H, H)),
        ],
        out_specs=_prow_spec(H),
        out_shape=jax.ShapeDtypeStruct((2, NP, H), jnp.float32),
    )(tab, agg, dinv, b1, w2, b2, gb, wa, wb, bu, gwn)


def _tc_pool_body(hp_ref, batch_ref, wpost, bpost, wro, bro, out_ref, acc):
    i = pl.program_id(0)

    @pl.when(i == 0)
    def _():
        acc[...] = jnp.zeros_like(acc)

    b = batch_ref[0, 0]
    brow = jnp.broadcast_to(b, (G, CH))
    gidx = lax.broadcasted_iota(jnp.int32, (G, CH), 0)
    mt = jnp.where(brow == gidx, 1.0, 0.0)
    acc[...] += _dot(mt, hp_ref[0])

    @pl.when(i == NP // CH - 1)
    def _():
        p = jax.nn.relu(_dot(acc[...], wpost[...]) + bpost[...])
        logits = _dot(p, wro[...]) + bro[...]
        m = jnp.max(logits, axis=1, keepdims=True)
        e = jnp.exp(logits - m)
        lse = jnp.log(jnp.sum(e, axis=1, keepdims=True)) + m
        out_ref[...] = logits - lse


def _tc_pool(tab, batch_r, wpost, bpost, wro, bro):
    return pl.pallas_call(
        _tc_pool_body,
        grid=(NP // CH,),
        in_specs=[
            pl.BlockSpec((1, CH, H), lambda i: (0, i, 0)),
            pl.BlockSpec((1, 1, CH), lambda i: (i, 0, 0)),
            _full_spec((H, H)), _full_spec((1, H)),
            _full_spec((H, NC)), _full_spec((1, NC)),
        ],
        out_specs=pl.BlockSpec((G, NC), lambda i: (0, 0)),
        out_shape=jax.ShapeDtypeStruct((G, NC), jnp.float32),
        scratch_shapes=[pltpu.VMEM((G, H), jnp.float32)],
    )(tab, batch_r, wpost, bpost, wro, bro)


@jax.jit
def _forward(x, stc_enc, edge_index, batch,
             W_pre, b_pre, W_s, b_s,
             gin_W1, gin_b1, gin_W2, gin_b2, gcn_W, gcn_b,
             W_hp, b_hp, W_post, b_post, W_ro, b_ro):
    f32 = jnp.float32
    src_r = jnp.pad(edge_index[0], (0, EP - E)).reshape(NSUB, NWA, CH)
    src2 = jnp.stack([src_r, src_r + NP])
    dst = jnp.pad(edge_index[1], (0, EP - E),
                  constant_values=DUMMY).reshape(NSUB, NWA, CH)
    xp = jnp.pad(x, ((0, NP - N), (0, 0)))
    stcp = jnp.pad(stc_enc, ((0, NP - N), (0, 0)))
    batch_r = jnp.pad(batch, (0, NP - N),
                      constant_values=G).reshape(NP // CH, 1, CH)

    z128 = jnp.zeros((STRIPE, H), f32)

    row = lambda v: v.reshape(1, -1)

    degw = _make_sc_deghist()(dst, jnp.zeros((NP // CH, CH), f32))

    u, xw = _tc_init(
        xp, stcp, W_pre, row(b_pre), W_s, row(b_s),
        gin_W1[0][:H], gin_W1[0][H:], gcn_W[0])
    dinv, tab = _tc_dinv(degw, u, xw)

    zero_bias = jnp.zeros((1, H), f32)
    for i in range(L):
        agg = _make_sc_agg()(tab.reshape(2 * NP, H), src2, dst, z128)
        last = i == L - 1
        if last:
            wa, wb, bu = W_hp[:H], W_hp[H:], row(b_hp)
            gwn = gcn_W[i]
        else:
            wa, wb, bu = gin_W1[i + 1][:H], gin_W1[i + 1][H:], zero_bias
            gwn = gcn_W[i + 1]
        tab = _tc_layer(
            tab, agg, dinv,
            row(gin_b1[i]), gin_W2[i], row(gin_b2[i]), row(gcn_b[i]),
            wa, wb, bu, gwn)

    return _tc_pool(tab, batch_r, W_post, row(b_post), W_ro, row(b_ro))


def kernel(x, stc_enc, edge_index, batch, y, W_pre, b_pre, W_s, b_s,
           gin_W1, gin_b1, gin_W2, gin_b2, gcn_W, gcn_b,
           W_hp, b_hp, W_post, b_post, W_ro, b_ro):
    return _forward(x, stc_enc, edge_index, batch,
                    W_pre, b_pre, W_s, b_s,
                    gin_W1, gin_b1, gin_W2, gin_b2, gcn_W, gcn_b,
                    W_hp, b_hp, W_post, b_post, W_ro, b_ro)


# speedup vs baseline: 7.3076x; 1.0587x over previous
import dataclasses
import functools

import jax
import jax.numpy as jnp
from jax import lax
from jax.experimental import pallas as pl
from jax.experimental.pallas import tpu as pltpu
from jax.experimental.pallas import tpu_sc as plsc

N = 10000
E = 320000
DF = 128
H = 128
NSE = 16
L = 3
G = 128
NC = 10

NSUB = 16
CH = 64
NWA = 320
NWD = NWA // 2
NB = 4
LN = 128
EP = NSUB * NWA * CH
NP = 10240
STRIPE = NP // NSUB
DUMMY = N


@functools.cache
def _mesh():
    return plsc.VectorSubcoreMesh(
        core_axis_name="c", subcore_axis_name="s", num_cores=2, num_subcores=16
    )


def _sc_deghist_body(dst_hbm, z_hbm, out, dstv, hist):
    c = lax.axis_index("c")
    s = lax.axis_index("s")
    pltpu.sync_copy(z_hbm, hist)
    pltpu.sync_copy(dst_hbm.at[s, pl.ds(c * NWD, NWD)], dstv)
    ones = jnp.ones((16,), jnp.float32)

    @pl.loop(0, NWD)
    def _(r):
        @pl.loop(0, CH, step=16)
        def _(j):
            d = dstv[r, pl.ds(j, 16)]
            plsc.addupdate_scatter(hist, [d >> 7, d & 127], ones)

    pltpu.sync_copy(hist, out.at[c, s])


@functools.cache
def _make_sc_deghist():
    return functools.partial(
        pl.kernel,
        out_type=jax.ShapeDtypeStruct((2, NSUB, NP // LN, LN), jnp.float32),
        mesh=_mesh(),
        scratch_types=[
            pltpu.VMEM((NWD, CH), jnp.int32),
            pltpu.VMEM((NP // LN, LN), jnp.float32),
        ],
        compiler_params=dataclasses.replace(pltpu.CompilerParams(),
                                            needs_layout_passes=False),
    )(_sc_deghist_body)


BLK = 40

def _sc_agg_body(tab_hbm, src_hbm, dst_hbm, z_hbm, out,
                 srcv, dstv, r0, r1, r2, r3, acc,
                 g0, g1, g2, g3, s0, s1, s2, s3):
    rows = (r0, r1, r2, r3)
    gsems = (g0, g1, g2, g3)
    ssems = (s0, s1, s2, s3)
    c = lax.axis_index("c")
    s = lax.axis_index("s")
    pltpu.sync_copy(z_hbm, acc.at[pl.ds(s * STRIPE, STRIPE)])
    plsc.subcore_barrier()

    for blk in range(NWA // BLK):
        pltpu.sync_copy(src_hbm.at[c, s, pl.ds(blk * BLK, BLK)], srcv)
        pltpu.sync_copy(dst_hbm.at[s, pl.ds(blk * BLK, BLK)], dstv)
        for b in range(NB):
            pltpu.async_copy(tab_hbm.at[srcv.at[b]], rows[b], gsems[b])

        @pl.loop(0, BLK - NB, step=NB)
        def _(g):
            for b in range(NB):
                k = g + b
                pltpu.make_async_copy(tab_hbm.at[srcv.at[b]], rows[b],
                                      gsems[b]).wait()
                pltpu.async_copy(rows[b], acc.at[dstv.at[k]], ssems[b],
                                 add=True)
                pltpu.make_async_copy(rows[b], acc.at[dstv.at[k]],
                                      ssems[b]).wait()
                pltpu.async_copy(tab_hbm.at[srcv.at[k + NB]], rows[b],
                                 gsems[b])

        for b in range(NB):
            k = BLK - NB + b
            pltpu.make_async_copy(tab_hbm.at[srcv.at[b]], rows[b],
                                  gsems[b]).wait()
            pltpu.sync_copy(rows[b], acc.at[dstv.at[k]], add=True)

    plsc.subcore_barrier()
    sl = pl.ds(s * STRIPE, STRIPE)
    pltpu.sync_copy(acc.at[sl], out.at[c, sl])


@functools.cache
def _make_sc_agg():
    return functools.partial(
        pl.kernel,
        out_type=jax.ShapeDtypeStruct((2, NP, H), jnp.float32),
        mesh=_mesh(),
        scratch_types=[
            pltpu.VMEM((BLK, CH), jnp.int32),
            pltpu.VMEM((BLK, CH), jnp.int32),
            pltpu.VMEM((CH, H), jnp.float32),
            pltpu.VMEM((CH, H), jnp.float32),
            pltpu.VMEM((CH, H), jnp.float32),
            pltpu.VMEM((CH, H), jnp.float32),
            pltpu.VMEM_SHARED((NP, H), jnp.float32),
            pltpu.SemaphoreType.DMA,
            pltpu.SemaphoreType.DMA,
            pltpu.SemaphoreType.DMA,
            pltpu.SemaphoreType.DMA,
            pltpu.SemaphoreType.DMA,
            pltpu.SemaphoreType.DMA,
            pltpu.SemaphoreType.DMA,
            pltpu.SemaphoreType.DMA,
        ],
    )(_sc_agg_body)


RB = 1280
_NG = NP // RB

def _dot(a, b):
    return jnp.dot(a, b, preferred_element_type=jnp.float32)


def _row_spec(w):
    return pl.BlockSpec((RB, w), lambda i: (i, 0))


def _prow_spec(w):
    return pl.BlockSpec((2, RB, w), lambda i: (0, i, 0))


def _full_spec(shape):
    return pl.BlockSpec(shape, lambda i: tuple(0 for _ in shape))


def _tc_init_body(x_ref, stc_ref, wpre, bpre, ws, bs, w1a, w1b, gw,
                  u_ref, xw_ref):
    h = _dot(x_ref[...], wpre[...]) + bpre[...]
    s = _dot(stc_ref[...], ws[...]) + bs[...]
    u_ref[...] = _dot(h, w1a[...]) + _dot(s, w1b[...])
    xw_ref[...] = _dot(s, gw[...])


def _tc_init(x, stc, wpre, bpre, ws, bs, w1a, w1b, gw):
    out = tuple(jax.ShapeDtypeStruct((NP, H), jnp.float32) for _ in range(2))
    return pl.pallas_call(
        _tc_init_body,
        grid=(_NG,),
        in_specs=[
            _row_spec(DF), _row_spec(NSE),
            _full_spec((DF, H)), _full_spec((1, H)),
            _full_spec((NSE, H)), _full_spec((1, H)),
            _full_spec((H, H)), _full_spec((H, H)), _full_spec((H, H)),
        ],
        out_specs=tuple(_row_spec(H) for _ in range(2)),
        out_shape=out,
    )(x, stc, wpre, bpre, ws, bs, w1a, w1b, gw)


def _tc_dinv_body(degw_ref, u_ref, xw_ref, dinv_ref, tab_ref):
    deg2d = jnp.sum(degw_ref[...], axis=(0, 1))
    dinv2d = lax.rsqrt(deg2d + 1.0)
    nsub = lax.broadcasted_iota(jnp.int32, (NP, NP // LN), 0) >> 7
    rsel = jnp.where(nsub == lax.broadcasted_iota(
        jnp.int32, (NP, NP // LN), 1), 1.0, 0.0)
    crow = _dot(rsel, dinv2d)
    nlan = lax.broadcasted_iota(jnp.int32, (NP, LN), 0) & (LN - 1)
    lsel = jnp.where(nlan == lax.broadcasted_iota(
        jnp.int32, (NP, LN), 1), 1.0, 0.0)
    dinv = jnp.sum(crow * lsel, axis=1, keepdims=True)
    dinv_ref[...] = jnp.broadcast_to(dinv, dinv_ref.shape)
    tab_ref[0] = u_ref[...]
    tab_ref[1] = dinv * xw_ref[...]


def _tc_dinv(degw, u, xw):
    return pl.pallas_call(
        _tc_dinv_body,
        grid=(1,),
        in_specs=[
            _full_spec((2, NSUB, NP // LN, LN)),
            _full_spec((NP, H)), _full_spec((NP, H)),
        ],
        out_specs=(_full_spec((NP, 16)), _full_spec((2, NP, H))),
        out_shape=(
            jax.ShapeDtypeStruct((NP, 16), jnp.float32),
            jax.ShapeDtypeStruct((2, NP, H), jnp.float32),
        ),
    )(degw, u, xw)


def _tc_layer_body(tab_ref, agg_ref, dinv_ref,
                   b1, w2, b2, gb, wa, wb, bu, gwn, tabn_ref):
    z = jax.nn.relu(tab_ref[0] + agg_ref[0] + b1[...])
    hn = jax.nn.relu(_dot(z, w2[...]) + b2[...])
    d = dinv_ref[...][:, 0:1]
    sn = jnp.tanh(d * (agg_ref[1] + tab_ref[1]) + gb[...])
    tabn_ref[0] = _dot(hn, wa[...]) + _dot(sn, wb[...]) + bu[...]
    tabn_ref[1] = d * _dot(sn, gwn[...])


def _tc_layer(tab, agg, dinv, b1, w2, b2, gb, wa, wb, bu, gwn):
    return pl.pallas_call(
        _tc_layer_body,
        grid=(_NG,),
        in_specs=[
            _prow_spec(H), _prow_spec(H), _row_spec(16),
            _full_spec((1, H)), _full_spec((H, H)), _full_spec((1, H)),
            _full_spec((1, H)),
            _full_spec((H, H)), _full_spec((H, H)), _full_spec((1, H)),
            _full_spec((H, H)),
        ],
        out_specs=_prow_spec(H),
        out_shape=jax.ShapeDtypeStruct((2, NP, H), jnp.float32),
    )(tab, agg, dinv, b1, w2, b2, gb, wa, wb, bu, gwn)


def _tc_pool_body(hp_ref, batch_ref, wpost, bpost, wro, bro, out_ref, acc):
    i = pl.program_id(0)

    @pl.when(i == 0)
    def _():
        acc[...] = jnp.zeros_like(acc)

    b = batch_ref[0, 0]
    brow = jnp.broadcast_to(b, (G, LN))
    gidx = lax.broadcasted_iota(jnp.int32, (G, LN), 0)
    mt = jnp.where(brow == gidx, 1.0, 0.0)
    acc[...] += _dot(mt, hp_ref[0])

    @pl.when(i == NP // LN - 1)
    def _():
        p = jax.nn.relu(_dot(acc[...], wpost[...]) + bpost[...])
        logits = _dot(p, wro[...]) + bro[...]
        m = jnp.max(logits, axis=1, keepdims=True)
        e = jnp.exp(logits - m)
        lse = jnp.log(jnp.sum(e, axis=1, keepdims=True)) + m
        out_ref[...] = logits - lse


def _tc_pool(tab, batch_r, wpost, bpost, wro, bro):
    return pl.pallas_call(
        _tc_pool_body,
        grid=(NP // LN,),
        in_specs=[
            pl.BlockSpec((1, LN, H), lambda i: (0, i, 0)),
            pl.BlockSpec((1, 1, LN), lambda i: (i, 0, 0)),
            _full_spec((H, H)), _full_spec((1, H)),
            _full_spec((H, NC)), _full_spec((1, NC)),
        ],
        out_specs=pl.BlockSpec((G, NC), lambda i: (0, 0)),
        out_shape=jax.ShapeDtypeStruct((G, NC), jnp.float32),
        scratch_shapes=[pltpu.VMEM((G, H), jnp.float32)],
    )(tab, batch_r, wpost, bpost, wro, bro)


@jax.jit
def _forward(x, stc_enc, edge_index, batch,
             W_pre, b_pre, W_s, b_s,
             gin_W1, gin_b1, gin_W2, gin_b2, gcn_W, gcn_b,
             W_hp, b_hp, W_post, b_post, W_ro, b_ro):
    f32 = jnp.float32
    src_r = jnp.pad(edge_index[0], (0, EP - E)).reshape(NSUB, NWA, CH)
    src2 = jnp.stack([src_r, src_r + NP])
    dst = jnp.pad(edge_index[1], (0, EP - E),
                  constant_values=DUMMY).reshape(NSUB, NWA, CH)
    xp = jnp.pad(x, ((0, NP - N), (0, 0)))
    stcp = jnp.pad(stc_enc, ((0, NP - N), (0, 0)))
    batch_r = jnp.pad(batch, (0, NP - N),
                      constant_values=G).reshape(NP // LN, 1, LN)

    z128 = jnp.zeros((STRIPE, H), f32)

    row = lambda v: v.reshape(1, -1)

    degw = _make_sc_deghist()(dst, jnp.zeros((NP // LN, LN), f32))

    u, xw = _tc_init(
        xp, stcp, W_pre, row(b_pre), W_s, row(b_s),
        gin_W1[0][:H], gin_W1[0][H:], gcn_W[0])
    dinv, tab = _tc_dinv(degw, u, xw)

    zero_bias = jnp.zeros((1, H), f32)
    for i in range(L):
        agg = _make_sc_agg()(tab.reshape(2 * NP, H), src2, dst, z128)
        last = i == L - 1
        if last:
            wa, wb, bu = W_hp[:H], W_hp[H:], row(b_hp)
            gwn = gcn_W[i]
        else:
            wa, wb, bu = gin_W1[i + 1][:H], gin_W1[i + 1][H:], zero_bias
            gwn = gcn_W[i + 1]
        tab = _tc_layer(
            tab, agg, dinv,
            row(gin_b1[i]), gin_W2[i], row(gin_b2[i]), row(gcn_b[i]),
            wa, wb, bu, gwn)

    return _tc_pool(tab, batch_r, W_post, row(b_post), W_ro, row(b_ro))


def kernel(x, stc_enc, edge_index, batch, y, W_pre, b_pre, W_s, b_s,
           gin_W1, gin_b1, gin_W2, gin_b2, gcn_W, gcn_b,
           W_hp, b_hp, W_post, b_post, W_ro, b_ro):
    return _forward(x, stc_enc, edge_index, batch,
                    W_pre, b_pre, W_s, b_s,
                    gin_W1, gin_b1, gin_W2, gin_b2, gcn_W, gcn_b,
                    W_hp, b_hp, W_post, b_post, W_ro, b_ro)
